# baseline (speedup 1.0000x reference)
import numpy as np
import jax
import jax.numpy as jnp
from jax import lax
from jax.experimental import pallas as pl
from jax.experimental.pallas import tpu as pltpu

IMG = 1024.0
A = 9
HF = 64
WF = 64
P = HF * WF
N = P * A
NR = N // 128
NMS_ITERS = 300
IOU_THR = 0.7
MIN_SCORE = 0.5


def _anchor_boxes():
    sizes = (32.0, 64.0, 128.0)
    ratios = (0.5, 1.0, 2.0)
    sy = IMG / HF
    sx = IMG / WF
    cy = (np.arange(HF) + 0.5) * sy
    cx = (np.arange(WF) + 0.5) * sx
    ahw = []
    for s in sizes:
        for r in ratios:
            ahw.append((s * np.sqrt(r), s / np.sqrt(r)))
    ahw = np.array(ahw, dtype=np.float64)
    CY, CX = np.meshgrid(cy, cx, indexing='ij')
    CY = CY[:, :, None]
    CX = CX[:, :, None]
    h = ahw[None, None, :, 0]
    w = ahw[None, None, :, 1]
    x1 = CX - 0.5 * w
    y1 = CY - 0.5 * h
    x2 = CX + 0.5 * w
    y2 = CY + 0.5 * h
    return jnp.asarray(
        np.stack([x1, y1, x2, y2], axis=-1).reshape(-1, 4), dtype=jnp.float32)


def _nms_body(x1_ref, y1_ref, x2_ref, y2_ref, s_ref,
              ox1_ref, oy1_ref, ox2_ref, oy2_ref,
              seff_ref, keep_ref, ar_ref):
    x1 = x1_ref[:]
    y1 = y1_ref[:]
    x2 = x2_ref[:]
    y2 = y2_ref[:]
    s = s_ref[:]
    bw = x2 - x1
    bh = y2 - y1
    areas = jnp.maximum(bw, 0.0) * jnp.maximum(bh, 0.0)
    ar_ref[:] = areas
    valid = (bw >= 1e-3) & (bh >= 1e-3) & (s >= MIN_SCORE)
    rows = lax.broadcasted_iota(jnp.int32, (NR, 128), 0)
    colx = lax.broadcasted_iota(jnp.int32, (NR, 128), 1)
    idxn = rows * 128 + colx
    lane = lax.broadcasted_iota(jnp.int32, (1, 128), 1)
    NEG = jnp.float32(-jnp.inf)
    seff_ref[:] = jnp.where(valid, s, NEG)
    keep_ref[:] = jnp.zeros((NR, 128), jnp.float32)

    def bred(a, op):
        n = a.shape[0]
        while n > 9:
            n //= 2
            a = op(a[:n], a[n:])
        a = op(a[:8], jnp.broadcast_to(a[8:9], (8, 128)))
        a = op(a[0:4], a[4:8])
        a = op(a[0:2], a[2:4])
        a = op(a[0:1], a[1:2])
        for sh in (64, 32, 16, 8, 4, 2, 1):
            a = op(a, pltpu.roll(a, sh, 1))
        return a

    def body(t, m):
        s_eff = seff_ref[:]
        eq = s_eff == m
        cand = jnp.where(eq, idxn, jnp.int32(N))
        i_row = bred(cand, jnp.minimum)
        sel = idxn == i_row

        def pick(a):
            return bred(jnp.where(sel, a, jnp.float32(-1.0)), jnp.maximum)

        x1v = x1_ref[:]
        y1v = y1_ref[:]
        x2v = x2_ref[:]
        y2v = y2_ref[:]
        av = ar_ref[:]
        xi1 = pick(x1v)
        yi1 = pick(y1v)
        xi2 = pick(x2v)
        yi2 = pick(y2v)
        ai = pick(av)
        ix1 = jnp.maximum(x1v, xi1)
        iy1 = jnp.maximum(y1v, yi1)
        ix2 = jnp.minimum(x2v, xi2)
        iy2 = jnp.minimum(y2v, yi2)
        inter = jnp.maximum(ix2 - ix1, 0.0) * jnp.maximum(iy2 - iy1, 0.0)
        iou = inter / (av + ai - inter + 1e-9)
        sup = iou > IOU_THR
        has = m > NEG
        s_new = jnp.where(jnp.logical_and(has, sup), NEG, s_eff)
        seff_ref[:] = s_new
        keep_ref[:] = jnp.where(jnp.logical_and(has, sel), 1.0, keep_ref[:])
        return bred(s_new, jnp.maximum)

    lax.fori_loop(0, NMS_ITERS, body, bred(seff_ref[:], jnp.maximum))
    k = keep_ref[:]
    ox1_ref[:] = x1 * k
    oy1_ref[:] = y1 * k
    ox2_ref[:] = x2 * k
    oy2_ref[:] = y2 * k


def _conv2d(x, w, b):
    y = lax.conv_general_dilated(x, w, (1, 1), 'SAME',
                                 dimension_numbers=('NCHW', 'OIHW', 'NCHW'))
    return y + b[None, :, None, None]


def kernel(feature_map, W1, b1, W2, b2, W3, b3):
    anchors = _anchor_boxes()
    x = jax.nn.relu(_conv2d(feature_map, W1, b1))
    cls_pred = jax.nn.sigmoid(_conv2d(x, W2, b2))
    off = _conv2d(x, W3, b3)
    B, _, H, W_ = off.shape
    off = jnp.transpose(off.reshape(B, A, 4, H, W_), (0, 3, 4, 1, 2)).reshape(-1, 4)
    scores = jnp.transpose(cls_pred, (0, 2, 3, 1)).reshape(-1)
    aw = anchors[:, 2] - anchors[:, 0]
    ah = anchors[:, 3] - anchors[:, 1]
    acx = anchors[:, 0] + 0.5 * aw
    acy = anchors[:, 1] + 0.5 * ah
    tx, ty, tw, th = off[:, 0], off[:, 1], off[:, 2], off[:, 3]
    cx = tx * aw + acx
    cy = ty * ah + acy
    w = jnp.exp(jnp.minimum(tw, 4.0)) * aw
    h = jnp.exp(jnp.minimum(th, 4.0)) * ah
    bx1 = jnp.clip(cx - 0.5 * w, 0.0, IMG)
    by1 = jnp.clip(cy - 0.5 * h, 0.0, IMG)
    bx2 = jnp.clip(cx + 0.5 * w, 0.0, IMG)
    by2 = jnp.clip(cy + 0.5 * h, 0.0, IMG)

    r = lambda a: a.reshape(NR, 128)
    fullb = pl.BlockSpec((NR, 128), lambda: (0, 0))
    ox1, oy1, ox2, oy2 = pl.pallas_call(
        _nms_body,
        in_specs=[fullb] * 5,
        out_specs=[fullb] * 4,
        out_shape=[jax.ShapeDtypeStruct((NR, 128), jnp.float32)] * 4,
        scratch_shapes=[pltpu.VMEM((NR, 128), jnp.float32),
                        pltpu.VMEM((NR, 128), jnp.float32),
                        pltpu.VMEM((NR, 128), jnp.float32)],
    )(r(bx1), r(by1), r(bx2), r(by2), r(scores))

    return jnp.stack([ox1.reshape(N), oy1.reshape(N),
                      ox2.reshape(N), oy2.reshape(N)], axis=-1)


# speedup vs baseline: 2.5658x; 2.5658x over previous
import numpy as np
import jax
import jax.numpy as jnp
from jax import lax
from jax.experimental import pallas as pl
from jax.experimental.pallas import tpu as pltpu

IMG = 1024.0
A = 9
HF = 64
WF = 64
P = HF * WF
N = P * A
NR = N // 128
NMS_ITERS = 300
IOU_THR = 0.7
MIN_SCORE = 0.5


def _anchor_boxes():
    sizes = (32.0, 64.0, 128.0)
    ratios = (0.5, 1.0, 2.0)
    sy = IMG / HF
    sx = IMG / WF
    cy = (np.arange(HF) + 0.5) * sy
    cx = (np.arange(WF) + 0.5) * sx
    ahw = []
    for s in sizes:
        for r in ratios:
            ahw.append((s * np.sqrt(r), s / np.sqrt(r)))
    ahw = np.array(ahw, dtype=np.float64)
    CY, CX = np.meshgrid(cy, cx, indexing='ij')
    CY = CY[:, :, None]
    CX = CX[:, :, None]
    h = ahw[None, None, :, 0]
    w = ahw[None, None, :, 1]
    x1 = CX - 0.5 * w
    y1 = CY - 0.5 * h
    x2 = CX + 0.5 * w
    y2 = CY + 0.5 * h
    return jnp.asarray(
        np.stack([x1, y1, x2, y2], axis=-1).reshape(-1, 4), dtype=jnp.float32)


def _nms_body(x1_ref, y1_ref, x2_ref, y2_ref, s_ref,
              ox1_ref, oy1_ref, ox2_ref, oy2_ref,
              seff_ref, keep_ref, ar_ref):
    x1 = x1_ref[:]
    y1 = y1_ref[:]
    x2 = x2_ref[:]
    y2 = y2_ref[:]
    s = s_ref[:]
    bw = x2 - x1
    bh = y2 - y1
    areas = jnp.maximum(bw, 0.0) * jnp.maximum(bh, 0.0)
    ar_ref[:] = areas
    valid = (bw >= 1e-3) & (bh >= 1e-3) & (s >= MIN_SCORE)
    rows = lax.broadcasted_iota(jnp.int32, (NR, 128), 0)
    colx = lax.broadcasted_iota(jnp.int32, (NR, 128), 1)
    idxn = rows * 128 + colx
    lane = lax.broadcasted_iota(jnp.int32, (1, 128), 1)
    NEG = jnp.float32(-jnp.inf)
    seff_ref[:] = jnp.where(valid, s, NEG)
    keep_ref[:] = jnp.zeros((NR, 128), jnp.float32)

    def argmax_tree(v):
        ia = idxn
        n = v.shape[0]
        while n > 9:
            n //= 2
            take = v[n:] > v[:n]
            v = jnp.where(take, v[n:], v[:n])
            ia = jnp.where(take, ia[n:], ia[:n])
        vt = jnp.broadcast_to(v[8:9], (8, 128))
        it = jnp.broadcast_to(ia[8:9], (8, 128))
        take = vt > v[:8]
        v = jnp.where(take, vt, v[:8])
        ia = jnp.where(take, it, ia[:8])
        m = jnp.max(v)
        i = jnp.min(jnp.where(v == m, ia, jnp.int32(N)))
        return m, i

    def body(t, carry):
        m, i = carry
        s_eff = seff_ref[:]
        r = i // 128
        lm = lane == (i % 128)

        def pick(ref):
            return jnp.max(jnp.where(lm, ref[pl.ds(r, 1), :],
                                     jnp.float32(-1.0)))

        x1v = x1_ref[:]
        y1v = y1_ref[:]
        x2v = x2_ref[:]
        y2v = y2_ref[:]
        av = ar_ref[:]
        xi1 = pick(x1_ref)
        yi1 = pick(y1_ref)
        xi2 = pick(x2_ref)
        yi2 = pick(y2_ref)
        ai = pick(ar_ref)
        ix1 = jnp.maximum(x1v, xi1)
        iy1 = jnp.maximum(y1v, yi1)
        ix2 = jnp.minimum(x2v, xi2)
        iy2 = jnp.minimum(y2v, yi2)
        inter = jnp.maximum(ix2 - ix1, 0.0) * jnp.maximum(iy2 - iy1, 0.0)
        iou = inter / (av + ai - inter + 1e-9)
        sup = iou > IOU_THR
        has = m > NEG
        s_new = jnp.where(jnp.logical_and(has, sup), NEG, s_eff)
        seff_ref[:] = s_new
        krow = keep_ref[pl.ds(r, 1), :]
        keep_ref[pl.ds(r, 1), :] = jnp.where(
            jnp.logical_and(has, lm), 1.0, krow)
        return argmax_tree(s_new)

    lax.fori_loop(0, NMS_ITERS, body, argmax_tree(seff_ref[:]))
    k = keep_ref[:]
    ox1_ref[:] = x1 * k
    oy1_ref[:] = y1 * k
    ox2_ref[:] = x2 * k
    oy2_ref[:] = y2 * k


def _conv2d(x, w, b):
    y = lax.conv_general_dilated(x, w, (1, 1), 'SAME',
                                 dimension_numbers=('NCHW', 'OIHW', 'NCHW'))
    return y + b[None, :, None, None]


def kernel(feature_map, W1, b1, W2, b2, W3, b3):
    anchors = _anchor_boxes()
    x = jax.nn.relu(_conv2d(feature_map, W1, b1))
    cls_pred = jax.nn.sigmoid(_conv2d(x, W2, b2))
    off = _conv2d(x, W3, b3)
    B, _, H, W_ = off.shape
    off = jnp.transpose(off.reshape(B, A, 4, H, W_), (0, 3, 4, 1, 2)).reshape(-1, 4)
    scores = jnp.transpose(cls_pred, (0, 2, 3, 1)).reshape(-1)
    aw = anchors[:, 2] - anchors[:, 0]
    ah = anchors[:, 3] - anchors[:, 1]
    acx = anchors[:, 0] + 0.5 * aw
    acy = anchors[:, 1] + 0.5 * ah
    tx, ty, tw, th = off[:, 0], off[:, 1], off[:, 2], off[:, 3]
    cx = tx * aw + acx
    cy = ty * ah + acy
    w = jnp.exp(jnp.minimum(tw, 4.0)) * aw
    h = jnp.exp(jnp.minimum(th, 4.0)) * ah
    bx1 = jnp.clip(cx - 0.5 * w, 0.0, IMG)
    by1 = jnp.clip(cy - 0.5 * h, 0.0, IMG)
    bx2 = jnp.clip(cx + 0.5 * w, 0.0, IMG)
    by2 = jnp.clip(cy + 0.5 * h, 0.0, IMG)

    r = lambda a: a.reshape(NR, 128)
    fullb = pl.BlockSpec((NR, 128), lambda: (0, 0))
    ox1, oy1, ox2, oy2 = pl.pallas_call(
        _nms_body,
        in_specs=[fullb] * 5,
        out_specs=[fullb] * 4,
        out_shape=[jax.ShapeDtypeStruct((NR, 128), jnp.float32)] * 4,
        scratch_shapes=[pltpu.VMEM((NR, 128), jnp.float32),
                        pltpu.VMEM((NR, 128), jnp.float32),
                        pltpu.VMEM((NR, 128), jnp.float32)],
    )(r(bx1), r(by1), r(bx2), r(by2), r(scores))

    return jnp.stack([ox1.reshape(N), oy1.reshape(N),
                      ox2.reshape(N), oy2.reshape(N)], axis=-1)
